# baseline (speedup 1.0000x reference)
import functools

import jax
import jax.numpy as jnp
from jax import lax
from jax.experimental import pallas as pl
from jax.experimental.pallas import tpu as pltpu
from jax.experimental.pallas import tpu_sc as plsc

N_NODE = 50000
E = 800000
DF = 128
H = 64
HH = 32
NC = 2
NS = 16
L = 16

EPT = E // NS
BLK = 80
NBLK = EPT // BLK
ROWS_PT = 3128
WCH = 184
NWCH = ROWS_PT // WCH
TAIL0 = (NS - 1) * ROWS_PT + (NWCH - 1) * WCH
TAILN = N_NODE - TAIL0

_BR = 2000
_GN = N_NODE // _BR
_BE = 2000
_GE = E // _BE


def _ileave(x):
    return jnp.stack([x[:, :L], x[:, L:]], axis=-1).reshape(x.shape[0], HH)


def _proj_body(x_ref, w_ref, b_ref, lo_ref, hi_ref, blo_ref, bhi_ref):
    h = jnp.dot(x_ref[...], w_ref[...], preferred_element_type=jnp.float32)
    h = jnp.maximum(h + b_ref[...], 0.0)
    lo_ref[...] = h[:, :HH]
    hi_ref[...] = h[:, HH:]
    blo_ref[...] = _ileave(h[:, :HH]).astype(jnp.bfloat16)
    bhi_ref[...] = _ileave(h[:, HH:]).astype(jnp.bfloat16)


def _proj(x, w, b):
    return pl.pallas_call(
        _proj_body,
        grid=(_GN,),
        in_specs=[
            pl.BlockSpec((_BR, DF), lambda i: (i, 0)),
            pl.BlockSpec((DF, H), lambda i: (0, 0)),
            pl.BlockSpec((1, H), lambda i: (0, 0)),
        ],
        out_specs=[pl.BlockSpec((_BR, HH), lambda i: (i, 0))] * 4,
        out_shape=[jax.ShapeDtypeStruct((N_NODE, HH), jnp.float32)] * 2
        + [jax.ShapeDtypeStruct((N_NODE, HH), jnp.bfloat16)] * 2,
    )(x, w, b.reshape(1, H))


def _edge_body(ea_ref, *refs):
    a = ea_ref[...]
    for k in range(4):
        w_ref, b_ref = refs[2 * k], refs[2 * k + 1]
        t = jnp.dot(a, w_ref[...], preferred_element_type=jnp.float32)
        t = t + b_ref[...]
        refs[8 + 2 * k][...] = _ileave(t[:, :HH]).astype(jnp.bfloat16)
        refs[8 + 2 * k + 1][...] = _ileave(t[:, HH:]).astype(jnp.bfloat16)


def _edge_terms(ea, Ws, bs):
    de = ea.shape[1]
    wb = []
    for w, b in zip(Ws, bs):
        wb += [w, b.reshape(1, H)]
    return pl.pallas_call(
        _edge_body,
        grid=(_GE,),
        in_specs=[pl.BlockSpec((_BE, de), lambda i: (i, 0))]
        + [pl.BlockSpec((de, H), lambda i: (0, 0)),
           pl.BlockSpec((1, H), lambda i: (0, 0))] * 4,
        out_specs=[pl.BlockSpec((_BE, HH), lambda i: (i, 0))] * 8,
        out_shape=[jax.ShapeDtypeStruct((E, HH), jnp.bfloat16)] * 8,
    )(ea, *wb)


def _mlp_body(residual, alo_ref, ahi_ref, hlo_ref, hhi_ref, sc_ref,
              w1_ref, b1_ref, w2_ref, b2_ref, *out_refs):
    h = jnp.concatenate([hlo_ref[...], hhi_ref[...]], axis=1)
    a = jnp.concatenate([alo_ref[...], ahi_ref[...]], axis=1)
    z = sc_ref[...] * h + a
    y = jnp.dot(z, w1_ref[...], preferred_element_type=jnp.float32)
    y = jnp.maximum(y + b1_ref[...], 0.0)
    o = jnp.dot(y, w2_ref[...], preferred_element_type=jnp.float32)
    o = jnp.maximum(o + b2_ref[...], 0.0)
    if residual:
        out_refs[0][...] = h + o
    else:
        out_refs[0][...] = o[:, :HH]
        out_refs[1][...] = o[:, HH:]
        out_refs[2][...] = _ileave(o[:, :HH]).astype(jnp.bfloat16)
        out_refs[3][...] = _ileave(o[:, HH:]).astype(jnp.bfloat16)


def _mlp(residual, a_lo, a_hi, h_lo, h_hi, eps, w1, b1, w2, b2):
    if residual:
        out_specs = [pl.BlockSpec((_BR, H), lambda i: (i, 0))]
        out_shape = [jax.ShapeDtypeStruct((N_NODE, H), jnp.float32)]
    else:
        out_specs = [pl.BlockSpec((_BR, HH), lambda i: (i, 0))] * 4
        out_shape = [jax.ShapeDtypeStruct((N_NODE, HH), jnp.float32)] * 2 \
            + [jax.ShapeDtypeStruct((N_NODE, HH), jnp.bfloat16)] * 2
    scale = (1.0 + eps) * jnp.ones((1, H), jnp.float32)
    res = pl.pallas_call(
        functools.partial(_mlp_body, residual),
        grid=(_GN,),
        in_specs=[pl.BlockSpec((_BR, HH), lambda i: (i, 0))] * 4
        + [pl.BlockSpec((1, H), lambda i: (0, 0))]
        + [pl.BlockSpec((H, H), lambda i: (0, 0)),
           pl.BlockSpec((1, H), lambda i: (0, 0)),
           pl.BlockSpec((H, H), lambda i: (0, 0)),
           pl.BlockSpec((1, H), lambda i: (0, 0))],
        out_specs=out_specs,
        out_shape=out_shape,
    )(a_lo, a_hi, h_lo, h_hi, scale, w1, b1.reshape(1, H), w2, b2.reshape(1, H))
    return res if not residual else res[0]


def _make_conv():
    mesh = plsc.VectorSubcoreMesh(core_axis_name="c", subcore_axis_name="s",
                                  num_cores=NC, num_subcores=NS)

    @functools.partial(
        pl.kernel,
        out_type=[jax.ShapeDtypeStruct((N_NODE, HH), jnp.float32),
                  jax.ShapeDtypeStruct((N_NODE, HH), jnp.float32)],
        mesh=mesh,
        scratch_types=[
            [pltpu.VMEM((BLK,), jnp.int32)] * 3,
            [pltpu.VMEM((BLK,), jnp.int32)] * 3,
            [pltpu.VMEM((BLK, L), jnp.int32)] * 2,
            [pltpu.VMEM((BLK, L), jnp.int32)] * 2,
            [pltpu.VMEM((BLK, HH), jnp.float32)] * 2,
            pltpu.VMEM((WCH, HH), jnp.float32),
            pltpu.VMEM_SHARED((N_NODE, HH), jnp.float32),
            [pltpu.SemaphoreType.DMA] * 3,
            [pltpu.SemaphoreType.DMA] * 2,
            [pltpu.SemaphoreType.DMA] * 2,
            [pltpu.SemaphoreType.DMA] * 2,
        ],
        compiler_params=pltpu.CompilerParams(use_tc_tiling_on_sc=False),
    )
    def conv(gi, si, h_lo, h_hi, t_lo, t_hi, out_lo, out_hi,
             gidx, sidx, rows, tv, msg, buf, acc, isem, gsem, tsem, ssem):
        c = lax.axis_index("c")
        s = lax.axis_index("s")

        nfull = jnp.where(s == NS - 1, NWCH - 1, NWCH)

        def zbody(r, carry):
            for k in range(HH // L):
                buf[r, pl.ds(k * L, L)] = jnp.zeros((L,), jnp.float32)
            return carry
        lax.fori_loop(0, WCH, zbody, 0)

        def zcopy(q, carry):
            r0 = pl.multiple_of(s * ROWS_PT + q * WCH, 8)
            pltpu.sync_copy(buf, acc.at[pl.ds(r0, WCH)])
            return carry
        lax.fori_loop(0, nfull, zcopy, 0)

        @pl.when(s == NS - 1)
        def _():
            pltpu.sync_copy(buf.at[pl.ds(0, TAILN)], acc.at[pl.ds(TAIL0, TAILN)])
        plsc.subcore_barrier()

        def e_of(b):
            return pl.multiple_of(s * EPT + b * BLK, L)

        def issue_idx(b, slot):
            e0 = e_of(b)
            pltpu.async_copy(gi.at[pl.ds(e0, BLK)], gidx[slot], isem[slot])
            pltpu.async_copy(si.at[pl.ds(e0, BLK)], sidx[slot], isem[slot])

        def wait_idx(slot):
            pltpu.make_async_copy(gi.at[pl.ds(0, BLK)], gidx[slot], isem[slot]).wait()
            pltpu.make_async_copy(si.at[pl.ds(0, BLK)], sidx[slot], isem[slot]).wait()

        def run(h_ref, t_ref):
            def issue_fetch(b, islot, dslot):
                pltpu.async_copy(h_ref.at[gidx[islot]], rows[dslot], gsem[dslot])
                pltpu.async_copy(t_ref.at[pl.ds(e_of(b), BLK)], tv[dslot], tsem[dslot])

            issue_idx(0, 0)
            wait_idx(0)
            issue_fetch(0, 0, 0)
            issue_idx(1, 1)

            def phase(b, dslot, islot):
                nxt = (islot + 1) % 3
                prv = (islot + 2) % 3

                @pl.when(b + 1 < NBLK)
                def _():
                    wait_idx(nxt)

                @pl.when(b >= 1)
                def _():
                    pltpu.make_async_copy(msg[1 - dslot], acc.at[sidx[prv]],
                                          ssem[1 - dslot]).wait()

                @pl.when(b + 1 < NBLK)
                def _():
                    issue_fetch(b + 1, nxt, 1 - dslot)

                @pl.when(b + 2 < NBLK)
                def _():
                    issue_idx(b + 2, prv)

                pltpu.make_async_copy(h_ref.at[gidx[islot]], rows[dslot],
                                      gsem[dslot]).wait()
                pltpu.make_async_copy(t_ref.at[pl.ds(0, BLK)], tv[dslot],
                                      tsem[dslot]).wait()

                @plsc.parallel_loop(0, BLK, unroll=8)
                def _(j):
                    h_pk = rows[dslot][j, :]
                    t_pk = tv[dslot][j, :]
                    bc = lambda v: lax.bitcast_convert_type(v, jnp.float32)
                    h0 = bc(h_pk << 16)
                    h1 = bc(h_pk & jnp.int32(-65536))
                    t0 = bc(t_pk << 16)
                    t1 = bc(t_pk & jnp.int32(-65536))
                    msg[dslot][j, pl.ds(0, L)] = jnp.maximum(h0 + t0, 0.0)
                    msg[dslot][j, pl.ds(L, L)] = jnp.maximum(h1 + t1, 0.0)

                pltpu.async_copy(msg[dslot], acc.at[sidx[islot]],
                                 ssem[dslot], add=True)

            assert NBLK % 6 == 1
            def group(g, carry):
                for u in range(6):
                    phase(g * 6 + u, u % 2, u % 3)
                return carry
            lax.fori_loop(0, NBLK // 6, group, 0)
            phase(NBLK - 1, 0, 0)
            pltpu.make_async_copy(msg[0], acc.at[sidx[0]], ssem[0]).wait()

        @pl.when(c == 0)
        def _():
            run(h_lo, t_lo)

        @pl.when(c == 1)
        def _():
            run(h_hi, t_hi)
        plsc.subcore_barrier()

        def wbody(q, carry):
            r0 = pl.multiple_of(s * ROWS_PT + q * WCH, 8)
            pltpu.sync_copy(acc.at[pl.ds(r0, WCH)], buf)

            @pl.when(c == 0)
            def _():
                pltpu.sync_copy(buf, out_lo.at[pl.ds(r0, WCH)])

            @pl.when(c == 1)
            def _():
                pltpu.sync_copy(buf, out_hi.at[pl.ds(r0, WCH)])
            return carry
        lax.fori_loop(0, nfull, wbody, 0)

        @pl.when(s == NS - 1)
        def _():
            pltpu.sync_copy(acc.at[pl.ds(TAIL0, TAILN)], buf.at[pl.ds(0, TAILN)])

            @pl.when(c == 0)
            def _():
                pltpu.sync_copy(buf.at[pl.ds(0, TAILN)], out_lo.at[pl.ds(TAIL0, TAILN)])

            @pl.when(c == 1)
            def _():
                pltpu.sync_copy(buf.at[pl.ds(0, TAILN)], out_hi.at[pl.ds(TAIL0, TAILN)])

    return conv


_conv = _make_conv()


def kernel(x_user, x_item, edge_index, edge_attr, Wpu, bpu, Wpi, bpi,
           l0f_eps, l0f_We, l0f_be, l0f_W1, l0f_b1, l0f_W2, l0f_b2,
           l0r_eps, l0r_We, l0r_be, l0r_W1, l0r_b1, l0r_W2, l0r_b2,
           l1f_eps, l1f_We, l1f_be, l1f_W1, l1f_b1, l1f_W2, l1f_b2,
           l1r_eps, l1r_We, l1r_be, l1r_W1, l1r_b1, l1r_W2, l1r_b2):
    ei = edge_index.astype(jnp.int32)
    src, dst = ei[0], ei[1]

    def _pk(a):
        return jax.lax.bitcast_convert_type(
            a.reshape(a.shape[0], L, 2), jnp.int32)

    hu_lo, hu_hi, hu_blo, hu_bhi = _proj(x_user, Wpu, bpu)
    hi_lo, hi_hi, hi_blo, hi_bhi = _proj(x_item, Wpi, bpi)

    (t0f_lo, t0f_hi, t0r_lo, t0r_hi,
     t1f_lo, t1f_hi, t1r_lo, t1r_hi) = _edge_terms(
        edge_attr,
        (l0f_We, l0r_We, l1f_We, l1r_We),
        (l0f_be, l0r_be, l1f_be, l1r_be))

    af_lo, af_hi = _conv(src, dst, _pk(hu_blo), _pk(hu_bhi),
                         _pk(t0f_lo), _pk(t0f_hi))
    ar_lo, ar_hi = _conv(dst, src, _pk(hi_blo), _pk(hi_bhi),
                         _pk(t0r_lo), _pk(t0r_hi))
    h1i_lo, h1i_hi, h1i_blo, h1i_bhi = _mlp(False, af_lo, af_hi, hi_lo, hi_hi,
                                            l0f_eps, l0f_W1, l0f_b1, l0f_W2, l0f_b2)
    h1u_lo, h1u_hi, h1u_blo, h1u_bhi = _mlp(False, ar_lo, ar_hi, hu_lo, hu_hi,
                                            l0r_eps, l0r_W1, l0r_b1, l0r_W2, l0r_b2)

    bf_lo, bf_hi = _conv(src, dst, _pk(h1u_blo), _pk(h1u_bhi),
                         _pk(t1f_lo), _pk(t1f_hi))
    br_lo, br_hi = _conv(dst, src, _pk(h1i_blo), _pk(h1i_bhi),
                         _pk(t1r_lo), _pk(t1r_hi))
    out_i = _mlp(True, bf_lo, bf_hi, h1i_lo, h1i_hi,
                 l1f_eps, l1f_W1, l1f_b1, l1f_W2, l1f_b2)
    out_u = _mlp(True, br_lo, br_hi, h1u_lo, h1u_hi,
                 l1r_eps, l1r_W1, l1r_b1, l1r_W2, l1r_b2)
    return (out_u, out_i)


# speedup vs baseline: 12.1911x; 12.1911x over previous
import functools

import jax
import jax.numpy as jnp
from jax import lax
from jax.experimental import pallas as pl
from jax.experimental.pallas import tpu as pltpu
from jax.experimental.pallas import tpu_sc as plsc

N_NODE = 50000
E = 800000
DF = 128
H = 64
HH = 32
NC = 2
NS = 16
L = 16

EPT = E // NS
BLK = 80
NBLK = EPT // BLK
ROWS_PT = 3128
WCH = 184
NWCH = ROWS_PT // WCH
TAIL0 = (NS - 1) * ROWS_PT + (NWCH - 1) * WCH
TAILN = N_NODE - TAIL0

_BR = 2000
_GN = N_NODE // _BR
_BE = 2000
_GE = E // _BE


def _pack32(x):
    a = lax.bitcast_convert_type(
        x[:, :L].astype(jnp.bfloat16).astype(jnp.float32), jnp.int32)
    b = lax.bitcast_convert_type(
        x[:, L:].astype(jnp.bfloat16).astype(jnp.float32), jnp.int32)
    return lax.shift_right_logical(a, 16) | (b & jnp.int32(-65536))


def _proj_body(x_ref, w_ref, b_ref, lo_ref, hi_ref, blo_ref, bhi_ref):
    h = jnp.dot(x_ref[...], w_ref[...], preferred_element_type=jnp.float32)
    h = jnp.maximum(h + b_ref[...], 0.0)
    lo_ref[...] = h[:, :HH]
    hi_ref[...] = h[:, HH:]
    blo_ref[...] = _pack32(h[:, :HH])
    bhi_ref[...] = _pack32(h[:, HH:])


def _proj(x, w, b):
    return pl.pallas_call(
        _proj_body,
        grid=(_GN,),
        in_specs=[
            pl.BlockSpec((_BR, DF), lambda i: (i, 0)),
            pl.BlockSpec((DF, H), lambda i: (0, 0)),
            pl.BlockSpec((1, H), lambda i: (0, 0)),
        ],
        out_specs=[pl.BlockSpec((_BR, HH), lambda i: (i, 0))] * 2
        + [pl.BlockSpec((_BR, L), lambda i: (i, 0))] * 2,
        out_shape=[jax.ShapeDtypeStruct((N_NODE, HH), jnp.float32)] * 2
        + [jax.ShapeDtypeStruct((N_NODE, L), jnp.int32)] * 2,
    )(x, w, b.reshape(1, H))


def _edge_body(ea_ref, *refs):
    a = ea_ref[...]
    for k in range(4):
        w_ref, b_ref = refs[2 * k], refs[2 * k + 1]
        t = jnp.dot(a, w_ref[...], preferred_element_type=jnp.float32)
        t = t + b_ref[...]
        refs[8 + 2 * k][...] = _pack32(t[:, :HH])
        refs[8 + 2 * k + 1][...] = _pack32(t[:, HH:])


def _edge_terms(ea, Ws, bs):
    de = ea.shape[1]
    wb = []
    for w, b in zip(Ws, bs):
        wb += [w, b.reshape(1, H)]
    return pl.pallas_call(
        _edge_body,
        grid=(_GE,),
        in_specs=[pl.BlockSpec((_BE, de), lambda i: (i, 0))]
        + [pl.BlockSpec((de, H), lambda i: (0, 0)),
           pl.BlockSpec((1, H), lambda i: (0, 0))] * 4,
        out_specs=[pl.BlockSpec((_BE, L), lambda i: (i, 0))] * 8,
        out_shape=[jax.ShapeDtypeStruct((E, L), jnp.int32)] * 8,
    )(ea, *wb)


def _mlp_body(residual, alo_ref, ahi_ref, hlo_ref, hhi_ref, sc_ref,
              w1_ref, b1_ref, w2_ref, b2_ref, *out_refs):
    h = jnp.concatenate([hlo_ref[...], hhi_ref[...]], axis=1)
    a = jnp.concatenate([alo_ref[...], ahi_ref[...]], axis=1)
    z = sc_ref[...] * h + a
    y = jnp.dot(z, w1_ref[...], preferred_element_type=jnp.float32)
    y = jnp.maximum(y + b1_ref[...], 0.0)
    o = jnp.dot(y, w2_ref[...], preferred_element_type=jnp.float32)
    o = jnp.maximum(o + b2_ref[...], 0.0)
    if residual:
        out_refs[0][...] = h + o
    else:
        out_refs[0][...] = o[:, :HH]
        out_refs[1][...] = o[:, HH:]
        out_refs[2][...] = _pack32(o[:, :HH])
        out_refs[3][...] = _pack32(o[:, HH:])


def _mlp(residual, a_lo, a_hi, h_lo, h_hi, eps, w1, b1, w2, b2):
    if residual:
        out_specs = [pl.BlockSpec((_BR, H), lambda i: (i, 0))]
        out_shape = [jax.ShapeDtypeStruct((N_NODE, H), jnp.float32)]
    else:
        out_specs = [pl.BlockSpec((_BR, HH), lambda i: (i, 0))] * 2 \
            + [pl.BlockSpec((_BR, L), lambda i: (i, 0))] * 2
        out_shape = [jax.ShapeDtypeStruct((N_NODE, HH), jnp.float32)] * 2 \
            + [jax.ShapeDtypeStruct((N_NODE, L), jnp.int32)] * 2
    scale = (1.0 + eps) * jnp.ones((1, H), jnp.float32)
    res = pl.pallas_call(
        functools.partial(_mlp_body, residual),
        grid=(_GN,),
        in_specs=[pl.BlockSpec((_BR, HH), lambda i: (i, 0))] * 4
        + [pl.BlockSpec((1, H), lambda i: (0, 0))]
        + [pl.BlockSpec((H, H), lambda i: (0, 0)),
           pl.BlockSpec((1, H), lambda i: (0, 0)),
           pl.BlockSpec((H, H), lambda i: (0, 0)),
           pl.BlockSpec((1, H), lambda i: (0, 0))],
        out_specs=out_specs,
        out_shape=out_shape,
    )(a_lo, a_hi, h_lo, h_hi, scale, w1, b1.reshape(1, H), w2, b2.reshape(1, H))
    return res if not residual else res[0]


def _make_conv():
    mesh = plsc.VectorSubcoreMesh(core_axis_name="c", subcore_axis_name="s",
                                  num_cores=NC, num_subcores=NS)

    @functools.partial(
        pl.kernel,
        out_type=[jax.ShapeDtypeStruct((N_NODE, HH), jnp.float32),
                  jax.ShapeDtypeStruct((N_NODE, HH), jnp.float32)],
        mesh=mesh,
        scratch_types=[
            [pltpu.VMEM((BLK,), jnp.int32)] * 3,
            [pltpu.VMEM((BLK,), jnp.int32)] * 3,
            [pltpu.VMEM((BLK, L), jnp.int32)] * 2,
            [pltpu.VMEM((BLK, L), jnp.int32)] * 2,
            [pltpu.VMEM((BLK, HH), jnp.float32)] * 2,
            pltpu.VMEM((WCH, HH), jnp.float32),
            pltpu.VMEM_SHARED((N_NODE, HH), jnp.float32),
            [pltpu.SemaphoreType.DMA] * 3,
            [pltpu.SemaphoreType.DMA] * 2,
            [pltpu.SemaphoreType.DMA] * 2,
            [pltpu.SemaphoreType.DMA] * 2,
        ],
        compiler_params=pltpu.CompilerParams(use_tc_tiling_on_sc=False),
    )
    def conv(gi, si, h_lo, h_hi, t_lo, t_hi, out_lo, out_hi,
             gidx, sidx, rows, tv, msg, buf, acc, isem, gsem, tsem, ssem):
        c = lax.axis_index("c")
        s = lax.axis_index("s")

        nfull = jnp.where(s == NS - 1, NWCH - 1, NWCH)

        def zbody(r, carry):
            for k in range(HH // L):
                buf[r, pl.ds(k * L, L)] = jnp.zeros((L,), jnp.float32)
            return carry
        lax.fori_loop(0, WCH, zbody, 0)

        def zcopy(q, carry):
            r0 = pl.multiple_of(s * ROWS_PT + q * WCH, 8)
            pltpu.sync_copy(buf, acc.at[pl.ds(r0, WCH)])
            return carry
        lax.fori_loop(0, nfull, zcopy, 0)

        @pl.when(s == NS - 1)
        def _():
            pltpu.sync_copy(buf.at[pl.ds(0, TAILN)], acc.at[pl.ds(TAIL0, TAILN)])
        plsc.subcore_barrier()

        def e_of(b):
            return pl.multiple_of(s * EPT + b * BLK, L)

        def issue_idx(b, slot):
            e0 = e_of(b)
            pltpu.async_copy(gi.at[pl.ds(e0, BLK)], gidx[slot], isem[slot])
            pltpu.async_copy(si.at[pl.ds(e0, BLK)], sidx[slot], isem[slot])

        def wait_idx(slot):
            pltpu.make_async_copy(gi.at[pl.ds(0, BLK)], gidx[slot], isem[slot]).wait()
            pltpu.make_async_copy(si.at[pl.ds(0, BLK)], sidx[slot], isem[slot]).wait()

        def run(h_ref, t_ref):
            def issue_fetch(b, islot, dslot):
                pltpu.async_copy(h_ref.at[gidx[islot]], rows[dslot], gsem[dslot])
                pltpu.async_copy(t_ref.at[pl.ds(e_of(b), BLK)], tv[dslot], tsem[dslot])

            issue_idx(0, 0)
            wait_idx(0)
            issue_fetch(0, 0, 0)
            issue_idx(1, 1)

            def phase(b, dslot, islot):
                nxt = (islot + 1) % 3
                prv = (islot + 2) % 3

                @pl.when(b + 1 < NBLK)
                def _():
                    wait_idx(nxt)

                @pl.when(b >= 1)
                def _():
                    pltpu.make_async_copy(msg[1 - dslot], acc.at[sidx[prv]],
                                          ssem[1 - dslot]).wait()

                @pl.when(b + 1 < NBLK)
                def _():
                    issue_fetch(b + 1, nxt, 1 - dslot)

                @pl.when(b + 2 < NBLK)
                def _():
                    issue_idx(b + 2, prv)

                pltpu.make_async_copy(h_ref.at[gidx[islot]], rows[dslot],
                                      gsem[dslot]).wait()
                pltpu.make_async_copy(t_ref.at[pl.ds(0, BLK)], tv[dslot],
                                      tsem[dslot]).wait()

                @plsc.parallel_loop(0, BLK, unroll=8)
                def _(j):
                    h_pk = rows[dslot][j, :]
                    t_pk = tv[dslot][j, :]
                    bc = lambda v: lax.bitcast_convert_type(v, jnp.float32)
                    h0 = bc(h_pk << 16)
                    h1 = bc(h_pk & jnp.int32(-65536))
                    t0 = bc(t_pk << 16)
                    t1 = bc(t_pk & jnp.int32(-65536))
                    msg[dslot][j, pl.ds(0, L)] = jnp.maximum(h0 + t0, 0.0)
                    msg[dslot][j, pl.ds(L, L)] = jnp.maximum(h1 + t1, 0.0)

                pltpu.async_copy(msg[dslot], acc.at[sidx[islot]],
                                 ssem[dslot], add=True)

            assert NBLK % 6 == 1
            def group(g, carry):
                for u in range(6):
                    phase(g * 6 + u, u % 2, u % 3)
                return carry
            lax.fori_loop(0, NBLK // 6, group, 0)
            phase(NBLK - 1, 0, 0)
            pltpu.make_async_copy(msg[0], acc.at[sidx[0]], ssem[0]).wait()

        @pl.when(c == 0)
        def _():
            run(h_lo, t_lo)

        @pl.when(c == 1)
        def _():
            run(h_hi, t_hi)
        plsc.subcore_barrier()

        def wbody(q, carry):
            r0 = pl.multiple_of(s * ROWS_PT + q * WCH, 8)
            pltpu.sync_copy(acc.at[pl.ds(r0, WCH)], buf)

            @pl.when(c == 0)
            def _():
                pltpu.sync_copy(buf, out_lo.at[pl.ds(r0, WCH)])

            @pl.when(c == 1)
            def _():
                pltpu.sync_copy(buf, out_hi.at[pl.ds(r0, WCH)])
            return carry
        lax.fori_loop(0, nfull, wbody, 0)

        @pl.when(s == NS - 1)
        def _():
            pltpu.sync_copy(acc.at[pl.ds(TAIL0, TAILN)], buf.at[pl.ds(0, TAILN)])

            @pl.when(c == 0)
            def _():
                pltpu.sync_copy(buf.at[pl.ds(0, TAILN)], out_lo.at[pl.ds(TAIL0, TAILN)])

            @pl.when(c == 1)
            def _():
                pltpu.sync_copy(buf.at[pl.ds(0, TAILN)], out_hi.at[pl.ds(TAIL0, TAILN)])

    return conv


_conv = _make_conv()


def kernel(x_user, x_item, edge_index, edge_attr, Wpu, bpu, Wpi, bpi,
           l0f_eps, l0f_We, l0f_be, l0f_W1, l0f_b1, l0f_W2, l0f_b2,
           l0r_eps, l0r_We, l0r_be, l0r_W1, l0r_b1, l0r_W2, l0r_b2,
           l1f_eps, l1f_We, l1f_be, l1f_W1, l1f_b1, l1f_W2, l1f_b2,
           l1r_eps, l1r_We, l1r_be, l1r_W1, l1r_b1, l1r_W2, l1r_b2):
    ei = edge_index.astype(jnp.int32)
    src, dst = ei[0], ei[1]

    hu_lo, hu_hi, hu_blo, hu_bhi = _proj(x_user, Wpu, bpu)
    hi_lo, hi_hi, hi_blo, hi_bhi = _proj(x_item, Wpi, bpi)

    (t0f_lo, t0f_hi, t0r_lo, t0r_hi,
     t1f_lo, t1f_hi, t1r_lo, t1r_hi) = _edge_terms(
        edge_attr,
        (l0f_We, l0r_We, l1f_We, l1r_We),
        (l0f_be, l0r_be, l1f_be, l1r_be))

    af_lo, af_hi = _conv(src, dst, hu_blo, hu_bhi, t0f_lo, t0f_hi)
    ar_lo, ar_hi = _conv(dst, src, hi_blo, hi_bhi, t0r_lo, t0r_hi)
    h1i_lo, h1i_hi, h1i_blo, h1i_bhi = _mlp(False, af_lo, af_hi, hi_lo, hi_hi,
                                            l0f_eps, l0f_W1, l0f_b1, l0f_W2, l0f_b2)
    h1u_lo, h1u_hi, h1u_blo, h1u_bhi = _mlp(False, ar_lo, ar_hi, hu_lo, hu_hi,
                                            l0r_eps, l0r_W1, l0r_b1, l0r_W2, l0r_b2)

    bf_lo, bf_hi = _conv(src, dst, h1u_blo, h1u_bhi, t1f_lo, t1f_hi)
    br_lo, br_hi = _conv(dst, src, h1i_blo, h1i_bhi, t1r_lo, t1r_hi)
    out_i = _mlp(True, bf_lo, bf_hi, h1i_lo, h1i_hi,
                 l1f_eps, l1f_W1, l1f_b1, l1f_W2, l1f_b2)
    out_u = _mlp(True, br_lo, br_hi, h1u_lo, h1u_hi,
                 l1r_eps, l1r_W1, l1r_b1, l1r_W2, l1r_b2)
    return (out_u, out_i)
